# baseline (speedup 1.0000x reference)
import jax
import jax.numpy as jnp
from jax.experimental import pallas as pl
from jax.experimental.pallas import tpu as pltpu

_N, _C = 100000, 80
_ROWS = 2000
_GRID = _N // _ROWS


def _qfl_body(pred_ref, lab_ref, sc_ref, out_ref):
    i = pl.program_id(0)
    x = pred_ref[...]
    lab = lab_ref[0, 0, :]
    s = sc_ref[0, 0, :]

    ax = jnp.abs(x)
    t = jnp.exp(-ax)
    l1p = jnp.log1p(t)
    relu = jnp.maximum(x, 0.0)
    sig = jnp.where(x >= 0, 1.0, t) / (1.0 + t)

    neg = (relu + l1p) * sig * sig
    sb = s[:, None]
    d = sb - sig
    pos = (relu - x * sb + l1p) * d * d

    col = jax.lax.broadcasted_iota(jnp.int32, x.shape, 1)
    m = col == lab[:, None]
    part = jnp.sum(jnp.where(m, pos, neg))

    @pl.when(i == 0)
    def _init():
        out_ref[0, 0] = part

    @pl.when(i > 0)
    def _acc():
        out_ref[0, 0] += part


def kernel(pred, label, score):
    lab3 = label.astype(jnp.int32).reshape(_GRID, 1, _ROWS)
    sc3 = score.reshape(_GRID, 1, _ROWS)
    total = pl.pallas_call(
        _qfl_body,
        grid=(_GRID,),
        in_specs=[
            pl.BlockSpec((_ROWS, _C), lambda i: (i, 0)),
            pl.BlockSpec((1, 1, _ROWS), lambda i: (i, 0, 0)),
            pl.BlockSpec((1, 1, _ROWS), lambda i: (i, 0, 0)),
        ],
        out_specs=pl.BlockSpec(memory_space=pltpu.SMEM),
        out_shape=jax.ShapeDtypeStruct((1, 1), jnp.float32),
    )(pred, lab3, sc3)
    return total[0, 0] / _N
